# baseline (speedup 1.0000x reference)
import functools

import jax
import jax.numpy as jnp
from jax import lax
from jax.experimental import pallas as pl
from jax.experimental.pallas import tpu as pltpu
from jax.experimental.pallas import tpu_sc as plsc

N = 10000
E = 320000
D = 128
L = 3
NF = 3
V = 8

NC = 2
NS = 16
NW = NC * NS
EPW = E // NW
K = 88
NCH = 120
EPWP = NCH * K
PADC = V ** NF
CPR = 1000

_mesh = plsc.VectorSubcoreMesh(core_axis_name="c", subcore_axis_name="s",
                               num_cores=NC, num_subcores=NS)


@functools.partial(
    pl.kernel,
    out_type=jax.ShapeDtypeStruct((NC, N, D), jnp.float32),
    mesh=_mesh,
    scratch_types=[
        pltpu.VMEM((8, 3, K), jnp.int32),
        pltpu.VMEM((2, K, D), jnp.float32),
        pltpu.VMEM((2, K, D), jnp.float32),
        pltpu.VMEM_SHARED((N, D), jnp.float32),
        pltpu.SemaphoreType.DMA((8,)),
        pltpu.SemaphoreType.DMA((2,)),
        pltpu.SemaphoreType.DMA((2,)),
        pltpu.SemaphoreType.DMA((2,)),
    ],
)
def _sc_message_pass(idx3_hbm, h_hbm, ctab_hbm, zeros_hbm, out_hbm,
                     idx_v, hrow_v, crow_v, agg_sh,
                     sem_i, sem_h, sem_c, sem_s):
    c = lax.axis_index("c")
    s = lax.axis_index("s")
    wid = c * NS + s

    @pl.when(s < N // CPR)
    def _zero():
        r0 = s * CPR
        pltpu.sync_copy(zeros_hbm.at[pl.ds(r0, CPR)], agg_sh.at[pl.ds(r0, CPR)])

    plsc.subcore_barrier()

    def idx_copy(i, slot):
        pltpu.async_copy(idx3_hbm.at[wid, i], idx_v.at[slot], sem_i.at[slot])

    def gathers(islot, db):
        pltpu.make_async_copy(idx3_hbm.at[wid, 0],
                              idx_v.at[islot], sem_i.at[islot]).wait()
        pltpu.async_copy(h_hbm.at[idx_v.at[islot, 0]], hrow_v.at[db],
                         sem_h.at[db])
        pltpu.async_copy(ctab_hbm.at[idx_v.at[islot, 1]], crow_v.at[db],
                         sem_c.at[db])

    for j in range(4):
        idx_copy(j, j)
    gathers(0, 0)

    def body(t, carry):
        for b in range(8):
            i = 8 * t + b
            db = b % 2

            def wait_scatter_prev():
                pltpu.make_async_copy(
                    crow_v.at[1 - db], agg_sh.at[idx_v.at[(b - 1) % 8, 2]],
                    sem_s.at[1 - db]).wait()
            if b > 0:
                wait_scatter_prev()
            else:
                pl.when(t > 0)(wait_scatter_prev)

            def pf_gather():
                gathers((b + 1) % 8, 1 - db)
            if b < 7:
                pf_gather()
            else:
                pl.when(t < NCH // 8 - 1)(pf_gather)

            pltpu.make_async_copy(h_hbm.at[idx_v.at[b, 0]],
                                  hrow_v.at[db], sem_h.at[db]).wait()
            pltpu.make_async_copy(ctab_hbm.at[idx_v.at[b, 1]],
                                  crow_v.at[db], sem_c.at[db]).wait()

            def row(e, carry2):
                for j in range(D // 16):
                    sl = pl.ds(j * 16, 16)
                    crow_v[db, e, sl] = jnp.maximum(
                        hrow_v[db, e, sl] + crow_v[db, e, sl], 0.0)
                return carry2

            lax.fori_loop(0, K, row, 0)

            def pf_idx():
                idx_copy(i + 4, (b + 4) % 8)
            if b < 4:
                pf_idx()
            else:
                pl.when(t < NCH // 8 - 1)(pf_idx)

            pltpu.async_copy(crow_v.at[db], agg_sh.at[idx_v.at[b, 2]],
                             sem_s.at[db], add=True)
        return carry

    lax.fori_loop(0, NCH // 8, body, 0)

    pltpu.make_async_copy(crow_v.at[1], agg_sh.at[idx_v.at[7, 2]],
                          sem_s.at[1]).wait()

    plsc.subcore_barrier()

    @pl.when(s < N // CPR)
    def _out():
        q0 = s * CPR
        pltpu.sync_copy(agg_sh.at[pl.ds(q0, CPR)], out_hbm.at[c, pl.ds(q0, CPR)])


def _mlp_body(h_ref, agg_ref, eps_ref, w1_ref, b1_ref, g1_ref, bb1_ref,
              w2_ref, b2_ref, g2_ref, bb2_ref, out_ref, *, final_relu):
    h = h_ref[...]
    pre = (1.0 + eps_ref[0, 0]) * h + agg_ref[0] + agg_ref[1]
    t = jnp.dot(pre, w1_ref[...], preferred_element_type=jnp.float32) + b1_ref[...]
    m = jnp.mean(t, axis=0, keepdims=True)
    v = jnp.mean((t - m) ** 2, axis=0, keepdims=True)
    t = (t - m) * lax.rsqrt(v + 1e-5) * g1_ref[...] + bb1_ref[...]
    t = jnp.maximum(t, 0.0)
    t2 = jnp.dot(t, w2_ref[...], preferred_element_type=jnp.float32) + b2_ref[...]
    m2 = jnp.mean(t2, axis=0, keepdims=True)
    v2 = jnp.mean((t2 - m2) ** 2, axis=0, keepdims=True)
    t2 = (t2 - m2) * lax.rsqrt(v2 + 1e-5) * g2_ref[...] + bb2_ref[...]
    if final_relu:
        t2 = jnp.maximum(t2, 0.0)
    out_ref[...] = t2


def _mlp(h, agg, eps_l, w1, b1, g1, bb1, w2, b2, g2, bb2, final_relu):
    return pl.pallas_call(
        functools.partial(_mlp_body, final_relu=final_relu),
        out_shape=jax.ShapeDtypeStruct((N, D), jnp.float32),
    )(h, agg, eps_l, w1, b1, g1, bb1, w2, b2, g2, bb2)


def kernel(x, edge_index, edge_attr, eps, W1, b1, bn1_g, bn1_b, W2, b2,
           bond_emb, bn_g, bn_b):
    src = edge_index[0]
    dst = edge_index[1]
    code = edge_attr[:, 0] + V * edge_attr[:, 1] + V * V * edge_attr[:, 2]
    npad = EPWP - EPW
    src_p = jnp.concatenate(
        [src.reshape(NW, EPW), jnp.zeros((NW, npad), jnp.int32)], axis=1)
    code_p = jnp.concatenate(
        [code.reshape(NW, EPW), jnp.full((NW, npad), PADC, jnp.int32)], axis=1)
    dst_p = jnp.concatenate(
        [dst.reshape(NW, EPW), jnp.zeros((NW, npad), jnp.int32)], axis=1)
    idx3 = (jnp.stack([src_p, code_p, dst_p], axis=1)
            .astype(jnp.int32)
            .reshape(NW, 3, NCH, K)
            .transpose(0, 2, 1, 3))
    ctab = (bond_emb[:, 2][:, :, None, None, :]
            + bond_emb[:, 1][:, None, :, None, :]
            + bond_emb[:, 0][:, None, None, :, :]).reshape(L, V ** NF, D)
    ctab = jnp.concatenate(
        [ctab, jnp.full((L, 1, D), -1e30, jnp.float32)], axis=1)
    zeros = jnp.zeros((N, D), jnp.float32)

    h = x
    for l in range(L):
        agg = _sc_message_pass(idx3, h, ctab[l], zeros)
        h = _mlp(h, agg, eps[l].reshape(1, 1),
                 W1[l], b1[l].reshape(1, 2 * D),
                 bn1_g[l].reshape(1, 2 * D), bn1_b[l].reshape(1, 2 * D),
                 W2[l], b2[l].reshape(1, D),
                 bn_g[l].reshape(1, D), bn_b[l].reshape(1, D),
                 final_relu=(l < L - 1))
    return h


# speedup vs baseline: 1.7890x; 1.7890x over previous
import functools

import jax
import jax.numpy as jnp
from jax import lax
from jax.experimental import pallas as pl
from jax.experimental.pallas import tpu as pltpu
from jax.experimental.pallas import tpu_sc as plsc

N = 10000
E = 320000
D = 128
L = 3
NF = 3
V = 8

NC = 2
NS = 16
NW = NC * NS
EPW = E // NW
K = 128
NCH = -(-EPW // K)
EPWP = NCH * K
AGG_ROWS = 10240
ZPR = AGG_ROWS // NS
CPR = 1000

_mesh = plsc.VectorSubcoreMesh(core_axis_name="c", subcore_axis_name="s",
                               num_cores=NC, num_subcores=NS)


@functools.partial(
    pl.kernel,
    out_type=jax.ShapeDtypeStruct((NC, N, D), jnp.float32),
    mesh=_mesh,
    scratch_types=[
        pltpu.VMEM((3, K), jnp.int32),
        pltpu.VMEM((K, D), jnp.float32),
        pltpu.VMEM((K, D), jnp.float32),
        pltpu.VMEM_SHARED((AGG_ROWS, D), jnp.float32),
        pltpu.SemaphoreType.DMA,
        pltpu.SemaphoreType.DMA,
    ],
)
def _sc_message_pass(idx3_hbm, h_hbm, ctab_hbm, zeros_hbm, out_hbm,
                     idx_v, hrow_v, crow_v, agg_sh, sem1, sem2):
    c = lax.axis_index("c")
    s = lax.axis_index("s")
    wid = c * NS + s

    r0 = s * ZPR
    pltpu.sync_copy(zeros_hbm.at[pl.ds(r0, ZPR)], agg_sh.at[pl.ds(r0, ZPR)])

    plsc.subcore_barrier()

    def chunk(i, carry):
        pltpu.sync_copy(idx3_hbm.at[wid, :, pl.ds(i * K, K)], idx_v)
        cp_h = pltpu.async_copy(h_hbm.at[idx_v.at[0]], hrow_v, sem1)
        cp_c = pltpu.async_copy(ctab_hbm.at[idx_v.at[1]], crow_v, sem2)
        cp_h.wait()
        cp_c.wait()

        def row(e, carry2):
            for j in range(D // 16):
                sl = pl.ds(j * 16, 16)
                hrow_v[e, sl] = jnp.maximum(hrow_v[e, sl] + crow_v[e, sl], 0.0)
            return carry2

        lax.fori_loop(0, K, row, 0)
        pltpu.sync_copy(hrow_v, agg_sh.at[idx_v.at[2]], add=True)
        return carry

    lax.fori_loop(0, NCH, chunk, 0)
    plsc.subcore_barrier()

    @pl.when(s < N // CPR)
    def _out():
        r0 = s * CPR
        pltpu.sync_copy(agg_sh.at[pl.ds(r0, CPR)], out_hbm.at[c, pl.ds(r0, CPR)])


def _mlp_body(h_ref, agg_ref, eps_ref, w1_ref, b1_ref, g1_ref, bb1_ref,
              w2_ref, b2_ref, g2_ref, bb2_ref, out_ref, *, final_relu):
    h = h_ref[...]
    pre = (1.0 + eps_ref[0, 0]) * h + agg_ref[0] + agg_ref[1]
    t = jnp.dot(pre, w1_ref[...], preferred_element_type=jnp.float32) + b1_ref[...]
    m = jnp.mean(t, axis=0, keepdims=True)
    v = jnp.mean((t - m) ** 2, axis=0, keepdims=True)
    t = (t - m) * lax.rsqrt(v + 1e-5) * g1_ref[...] + bb1_ref[...]
    t = jnp.maximum(t, 0.0)
    t2 = jnp.dot(t, w2_ref[...], preferred_element_type=jnp.float32) + b2_ref[...]
    m2 = jnp.mean(t2, axis=0, keepdims=True)
    v2 = jnp.mean((t2 - m2) ** 2, axis=0, keepdims=True)
    t2 = (t2 - m2) * lax.rsqrt(v2 + 1e-5) * g2_ref[...] + bb2_ref[...]
    if final_relu:
        t2 = jnp.maximum(t2, 0.0)
    out_ref[...] = t2


def _mlp(h, agg, eps_l, w1, b1, g1, bb1, w2, b2, g2, bb2, final_relu):
    return pl.pallas_call(
        functools.partial(_mlp_body, final_relu=final_relu),
        out_shape=jax.ShapeDtypeStruct((N, D), jnp.float32),
    )(h, agg, eps_l, w1, b1, g1, bb1, w2, b2, g2, bb2)


def kernel(x, edge_index, edge_attr, eps, W1, b1, bn1_g, bn1_b, W2, b2,
           bond_emb, bn_g, bn_b):
    src = edge_index[0]
    dst = edge_index[1]
    code = edge_attr[:, 0] + V * edge_attr[:, 1] + V * V * edge_attr[:, 2]
    npad = EPWP - EPW
    src_p = jnp.concatenate(
        [src.reshape(NW, EPW), jnp.zeros((NW, npad), jnp.int32)], axis=1)
    code_p = jnp.concatenate(
        [code.reshape(NW, EPW), jnp.zeros((NW, npad), jnp.int32)], axis=1)
    dst_p = jnp.concatenate(
        [dst.reshape(NW, EPW), jnp.full((NW, npad), N, jnp.int32)], axis=1)
    idx3 = jnp.stack([src_p, code_p, dst_p], axis=1).astype(jnp.int32)
    ctab = (bond_emb[:, 2][:, :, None, None, :]
            + bond_emb[:, 1][:, None, :, None, :]
            + bond_emb[:, 0][:, None, None, :, :]).reshape(L, V ** NF, D)
    zeros = jnp.zeros((AGG_ROWS, D), jnp.float32)

    h = x
    for l in range(L):
        agg = _sc_message_pass(idx3, h, ctab[l], zeros)
        h = _mlp(h, agg, eps[l].reshape(1, 1),
                 W1[l], b1[l].reshape(1, 2 * D),
                 bn1_g[l].reshape(1, 2 * D), bn1_b[l].reshape(1, 2 * D),
                 W2[l], b2[l].reshape(1, D),
                 bn_g[l].reshape(1, D), bn_b[l].reshape(1, D),
                 final_relu=(l < L - 1))
    return h
